# baseline (speedup 1.0000x reference)
import functools

import jax
import jax.numpy as jnp
from jax import lax
from jax.experimental import pallas as pl
from jax.experimental.pallas import tpu as pltpu
from jax.experimental.pallas import tpu_sc as plsc

F = 26
V = 100000
D = 32
B = 16384
BF = B * F
NC, NS, L = 2, 16, 16
NW = NC * NS
PER_W = BF // NW
C = 1024
NCH = PER_W // C
VPC = C // L
NPAT = 13


def _body(idx_hbm, tab_hbm, out_hbm, idx_v, offs_v, buf0, buf1,
          gsem0, gsem1, ssem0, ssem1):
    wid = lax.axis_index("s") * NC + lax.axis_index("c")
    base = wid * PER_W

    pltpu.sync_copy(idx_hbm.at[pl.ds(base, PER_W)], idx_v)

    for t in range(NPAT):
        p = lax.iota(jnp.int32, L) + t * L
        offs_v[pl.ds(t * L, L)] = (p % F) * V

    def _prep(j):
        def vbody(v, _):
            c = j * C + v * L
            t = ((j * VPC + v) % NPAT) * L
            idx_v[pl.ds(c, L)] = idx_v[pl.ds(c, L)] + offs_v[pl.ds(t, L)]
            return 0
        lax.fori_loop(0, VPC, vbody, 0)

    bufs = (buf0, buf1)
    gsems = (gsem0, gsem1)
    ssems = (ssem0, ssem1)

    def _gather(j):
        return pltpu.async_copy(tab_hbm.at[idx_v.at[pl.ds(j * C, C)]],
                                bufs[j % 2], gsems[j % 2])

    def _store(j):
        return pltpu.async_copy(bufs[j % 2],
                                out_hbm.at[pl.ds(base + j * C, C)],
                                ssems[j % 2])

    _prep(0)
    gathers = {0: _gather(0)}
    _prep(1)
    stores = {}
    for j in range(NCH):
        if j + 1 < NCH:
            if j >= 1:
                stores[j - 1].wait()
            gathers[j + 1] = _gather(j + 1)
        gathers[j].wait()
        stores[j] = _store(j)
        if j + 2 < NCH:
            _prep(j + 2)
    stores[NCH - 2].wait()
    stores[NCH - 1].wait()


@jax.jit
def _run(idx, flat_tables):
    mesh = plsc.VectorSubcoreMesh(core_axis_name="c", subcore_axis_name="s")
    kfn = pl.kernel(
        _body,
        out_type=jax.ShapeDtypeStruct((BF, D), jnp.float32),
        mesh=mesh,
        compiler_params=pltpu.CompilerParams(use_tc_tiling_on_sc=False),
        scratch_types=[
            pltpu.VMEM((PER_W,), jnp.int32),
            pltpu.VMEM((NPAT * L,), jnp.int32),
            pltpu.VMEM((C, D), jnp.float32),
            pltpu.VMEM((C, D), jnp.float32),
            pltpu.SemaphoreType.DMA,
            pltpu.SemaphoreType.DMA,
            pltpu.SemaphoreType.DMA,
            pltpu.SemaphoreType.DMA,
        ],
    )
    return kfn(idx, flat_tables)


def kernel(tensor, tables):
    idx = tensor.astype(jnp.int32).reshape(BF)
    flat_tables = tables.reshape(F * V, D)
    out = _run(idx, flat_tables)
    return out.reshape(B, F * D)
